# baseline (speedup 1.0000x reference)
import functools

import jax
import jax.numpy as jnp
from jax import lax
from jax.experimental import pallas as pl
from jax.experimental.pallas import tpu as pltpu
from jax.experimental.pallas import tpu_sc as plsc

D_MODEL = 64
SCALE = 8.0
NUM_WORKERS = 32
CHUNK = 128
NBUF = 4


@functools.lru_cache(maxsize=None)
def _build(n_rows: int):
    rows_per_w = n_rows // NUM_WORKERS
    n_chunks = rows_per_w // CHUNK
    assert n_chunks % NBUF == 0

    mesh = plsc.VectorSubcoreMesh(core_axis_name="c", subcore_axis_name="s")

    scratch = [pltpu.VMEM((n_chunks, CHUNK), jnp.int32)]
    scratch += [pltpu.VMEM((CHUNK, D_MODEL), jnp.float32) for _ in range(NBUF)]
    scratch += [pltpu.SemaphoreType.DMA for _ in range(2 * NBUF)]

    @functools.partial(
        pl.kernel,
        mesh=mesh,
        out_type=jax.ShapeDtypeStruct(
            (NUM_WORKERS, n_chunks, CHUNK, D_MODEL), jnp.float32
        ),
        scratch_types=scratch,
        compiler_params=pltpu.CompilerParams(use_tc_tiling_on_sc=False),
    )
    def emb_kernel(x_hbm, tab_hbm, out_hbm, idx_v, *rest):
        bufs = rest[:NBUF]
        gsems = rest[NBUF : 2 * NBUF]
        osems = rest[2 * NBUF :]
        wid = lax.axis_index("s") * 2 + lax.axis_index("c")

        pltpu.sync_copy(x_hbm.at[wid], idx_v)

        for b in range(NBUF):
            pltpu.async_copy(tab_hbm.at[idx_v.at[b]], bufs[b], gsems[b])

        def outer(g, carry):
            for b in range(NBUF):
                j = g * NBUF + b
                pltpu.make_async_copy(
                    tab_hbm.at[idx_v.at[j]], bufs[b], gsems[b]
                ).wait()

                def scale(r, c, buf=bufs[b]):
                    for rr in range(4):
                        for u in range(4):
                            sl = (r * 4 + rr, pl.ds(u * 16, 16))
                            buf[sl] = buf[sl] * SCALE
                    return c

                lax.fori_loop(0, CHUNK // 4, scale, 0)

                pltpu.async_copy(bufs[b], out_hbm.at[wid, j], osems[b])
                nxt = j + NBUF

                @pl.when(nxt < n_chunks)
                def _(b=b, j=j, nxt=nxt):
                    pltpu.make_async_copy(
                        bufs[b], out_hbm.at[wid, j], osems[b]
                    ).wait()
                    pltpu.async_copy(
                        tab_hbm.at[idx_v.at[nxt]], bufs[b], gsems[b]
                    )

            return carry

        lax.fori_loop(0, n_chunks // NBUF, outer, 0)

        for b in range(NBUF):
            pltpu.make_async_copy(bufs[b], out_hbm.at[wid, 0], osems[b]).wait()

    return emb_kernel


def kernel(x, table):
    b, t = x.shape
    n_rows = b * t
    xr = x.reshape(NUM_WORKERS, n_rows // (NUM_WORKERS * CHUNK), CHUNK).astype(
        jnp.int32
    )
    out = _build(n_rows)(xr, table)
    return out.reshape(b, t, D_MODEL)
